# Initial kernel scaffold; baseline (speedup 1.0000x reference)
import jax
import jax.numpy as jnp
from jax.experimental import pallas as pl


def kernel(edge_index, h, e, params):
    raise NotImplementedError("write your pallas kernel here")


# speedup vs baseline: 1.2596x; 1.2596x over previous
import functools

import jax
import jax.numpy as jnp
from jax import lax
from jax.experimental import pallas as pl
from jax.experimental.pallas import tpu as pltpu
from jax.experimental.pallas import tpu_sc as plsc

N = 10000
E = 320000
D = 128

NC = 2
NS = 16
NW = NC * NS
EPW = E // NW
CH = 80
NCHUNK = EPW // CH
CH2 = 40
NCHUNK2 = EPW // CH2
ZCH = 40
NZC = N // ZCH
NLV = D // 16

_mesh = plsc.VectorSubcoreMesh(core_axis_name="c", subcore_axis_name="s")


def _node_mm_body(h_ref, w_ref, b_ref, t1_ref, t2_ref, u_ref):
    acc = jnp.dot(h_ref[...], w_ref[...], preferred_element_type=jnp.float32)
    acc = acc + b_ref[...]
    t1_ref[...] = acc[:, :2 * D]
    t2_ref[...] = acc[:, 2 * D:4 * D]
    u_ref[...] = acc[:, 4 * D:]


def _node_mm(h, wt, bb):
    bn = 2000
    return pl.pallas_call(
        _node_mm_body,
        grid=(N // bn,),
        in_specs=[
            pl.BlockSpec((bn, D), lambda i: (i, 0)),
            pl.BlockSpec((D, 5 * D), lambda i: (0, 0)),
            pl.BlockSpec((1, 5 * D), lambda i: (0, 0)),
        ],
        out_specs=[
            pl.BlockSpec((bn, 2 * D), lambda i: (i, 0)),
            pl.BlockSpec((bn, 2 * D), lambda i: (i, 0)),
            pl.BlockSpec((bn, D), lambda i: (i, 0)),
        ],
        out_shape=[
            jax.ShapeDtypeStruct((N, 2 * D), jnp.float32),
            jax.ShapeDtypeStruct((N, 2 * D), jnp.float32),
            jax.ShapeDtypeStruct((N, D), jnp.float32),
        ],
    )(h, wt, bb)


def _edge_mm_body(e_ref, w_ref, b_ref, c_ref):
    c_ref[...] = jnp.dot(e_ref[...], w_ref[...],
                         preferred_element_type=jnp.float32) + b_ref[...]


def _edge_mm(e, wt, bb):
    bn = 16000
    return pl.pallas_call(
        _edge_mm_body,
        grid=(E // bn,),
        in_specs=[
            pl.BlockSpec((bn, D), lambda i: (i, 0)),
            pl.BlockSpec((D, D), lambda i: (0, 0)),
            pl.BlockSpec((1, D), lambda i: (0, 0)),
        ],
        out_specs=pl.BlockSpec((bn, D), lambda i: (i, 0)),
        out_shape=jax.ShapeDtypeStruct((E, D), jnp.float32),
    )(e, wt, bb)


def _node_bn_body(hp_ref, u_ref, hin_ref, g_ref, b_ref, out_ref):
    s = hp_ref[0] + hp_ref[1] + u_ref[...]
    mu = jnp.mean(s, axis=0, keepdims=True)
    var = jnp.mean((s - mu) ** 2, axis=0, keepdims=True)
    y = (s - mu) * lax.rsqrt(var + 1e-5) * g_ref[...] + b_ref[...]
    out_ref[...] = jnp.maximum(y, 0.0) + hin_ref[...]


def _node_bn(hp, u, hin, g, b):
    return pl.pallas_call(
        _node_bn_body,
        out_shape=jax.ShapeDtypeStruct((N, D), jnp.float32),
    )(hp, u, hin, g, b)


def _pass1_body(t1_hbm, src_hbm, dst_hbm, c_hbm,
                xf_hbm, xb_hbm, stats_hbm,
                idx_s, idx_d, rows_s, rows_d, c_v, xf_v, xb_v, stats_v,
                sem0, sem1):
    cid = lax.axis_index("c")
    sid = lax.axis_index("s")
    wid = sid * NC + cid

    z = jnp.zeros((16,), jnp.float32)
    for r in range(4):
        for c in range(NLV):
            stats_v[r, pl.ds(c * 16, 16)] = z

    @pl.loop(0, NCHUNK)
    def _chunk(i):
        base = wid * EPW + i * CH
        pltpu.sync_copy(src_hbm.at[pl.ds(base, CH)], idx_s)
        pltpu.sync_copy(dst_hbm.at[pl.ds(base, CH)], idx_d)
        g1 = pltpu.async_copy(t1_hbm.at[idx_s], rows_s, sem0)
        g2 = pltpu.async_copy(t1_hbm.at[idx_d], rows_d, sem1)
        pltpu.sync_copy(c_hbm.at[pl.ds(base, CH)], c_v)
        g1.wait()
        g2.wait()

        def row(j, carry):
            acc = list(carry)
            for c in range(NLV):
                sl = pl.ds(c * 16, 16)
                sl2 = pl.ds(D + c * 16, 16)
                a_s = rows_s[j, sl]
                b_s = rows_s[j, sl2]
                a_d = rows_d[j, sl]
                b_d = rows_d[j, sl2]
                cc = c_v[j, sl]
                xf = a_s + b_d + cc
                xb = a_d + b_s + cc
                xf_v[j, sl] = xf
                xb_v[j, sl] = xb
                acc[c] = acc[c] + xf
                acc[NLV + c] = acc[NLV + c] + xf * xf
                acc[2 * NLV + c] = acc[2 * NLV + c] + xb
                acc[3 * NLV + c] = acc[3 * NLV + c] + xb * xb
            return tuple(acc)

        carry = lax.fori_loop(0, CH, row, (z,) * (4 * NLV))
        for c in range(NLV):
            sl = pl.ds(c * 16, 16)
            stats_v[0, sl] += carry[c]
            stats_v[1, sl] += carry[NLV + c]
            stats_v[2, sl] += carry[2 * NLV + c]
            stats_v[3, sl] += carry[3 * NLV + c]
        pltpu.sync_copy(xf_v, xf_hbm.at[pl.ds(base, CH)])
        pltpu.sync_copy(xb_v, xb_hbm.at[pl.ds(base, CH)])

    pltpu.sync_copy(stats_v, stats_hbm.at[wid])


_pass1 = functools.partial(
    pl.kernel,
    out_type=(
        jax.ShapeDtypeStruct((E, D), jnp.float32),
        jax.ShapeDtypeStruct((E, D), jnp.float32),
        jax.ShapeDtypeStruct((NW, 4, D), jnp.float32),
    ),
    mesh=_mesh,
    scratch_types=[
        pltpu.VMEM((CH,), jnp.int32),
        pltpu.VMEM((CH,), jnp.int32),
        pltpu.VMEM((CH, 2 * D), jnp.float32),
        pltpu.VMEM((CH, 2 * D), jnp.float32),
        pltpu.VMEM((CH, D), jnp.float32),
        pltpu.VMEM((CH, D), jnp.float32),
        pltpu.VMEM((CH, D), jnp.float32),
        pltpu.VMEM((4, D), jnp.float32),
        pltpu.SemaphoreType.DMA,
        pltpu.SemaphoreType.DMA,
    ],
)(_pass1_body)


def _pass2_body(t2_hbm, src_hbm, dst_hbm, xf_hbm, xb_hbm, e_hbm, ss_hbm,
                nef_hbm, hp_hbm,
                idx_s, idx_d, rows2, xf_v, xb_v, e_v, ss_v, zb,
                acc_sh, sem0, sem1):
    cid = lax.axis_index("c")
    sid = lax.axis_index("s")
    wid = sid * NC + cid

    z = jnp.zeros((16,), jnp.float32)

    def zrow(j, _):
        for c in range(NLV):
            zb[j, pl.ds(c * 16, 16)] = z
        return 0

    lax.fori_loop(0, ZCH, zrow, 0)
    for k in range((NZC + NS - 1) // NS):
        ci = sid + k * NS

        @pl.when(ci < NZC)
        def _():
            pltpu.sync_copy(zb, acc_sh.at[pl.ds(ci * ZCH, ZCH)])

    pltpu.sync_copy(ss_hbm, ss_v)
    plsc.subcore_barrier()

    scf = [ss_v[0, pl.ds(c * 16, 16)] for c in range(NLV)]
    shf = [ss_v[1, pl.ds(c * 16, 16)] for c in range(NLV)]
    scb = [ss_v[2, pl.ds(c * 16, 16)] for c in range(NLV)]
    shb = [ss_v[3, pl.ds(c * 16, 16)] for c in range(NLV)]

    @pl.loop(0, NCHUNK2)
    def _chunk(i):
        base = wid * EPW + i * CH2
        pltpu.sync_copy(src_hbm.at[pl.ds(base, CH2)], idx_s)
        pltpu.sync_copy(dst_hbm.at[pl.ds(base, CH2)], idx_d)
        g1 = pltpu.async_copy(t2_hbm.at[idx_s], rows2, sem0)
        pltpu.sync_copy(xf_hbm.at[pl.ds(base, CH2)], xf_v)
        pltpu.sync_copy(xb_hbm.at[pl.ds(base, CH2)], xb_v)
        pltpu.sync_copy(e_hbm.at[pl.ds(base, CH2)], e_v)
        g1.wait()

        def row(j, _):
            for c in range(NLV):
                sl = pl.ds(c * 16, 16)
                sl2 = pl.ds(D + c * 16, 16)
                ee = e_v[j, sl]
                nf = jnp.maximum(xf_v[j, sl] * scf[c] + shf[c], 0.0) + ee
                nb = jnp.maximum(xb_v[j, sl] * scb[c] + shb[c], 0.0) + ee
                xf_v[j, sl] = nf
                gf = 1.0 / (1e-6 * jnp.exp(-nf) + (1.0 + 1e-6))
                gb = 1.0 / (1e-6 * jnp.exp(-nb) + (1.0 + 1e-6))
                e_v[j, sl] = rows2[j, sl] * gf + rows2[j, sl2] * gb
            return 0

        lax.fori_loop(0, CH2, row, 0)
        pltpu.sync_copy(xf_v, nef_hbm.at[pl.ds(base, CH2)])
        pltpu.sync_copy(e_v, acc_sh.at[idx_d], add=True)

    plsc.subcore_barrier()
    for k in range((NZC + NS - 1) // NS):
        ci = sid + k * NS

        @pl.when(ci < NZC)
        def _():
            r0 = ci * ZCH
            pltpu.sync_copy(acc_sh.at[pl.ds(r0, ZCH)],
                            hp_hbm.at[cid, pl.ds(r0, ZCH)])


_pass2 = functools.partial(
    pl.kernel,
    out_type=(
        jax.ShapeDtypeStruct((E, D), jnp.float32),
        jax.ShapeDtypeStruct((NC, N, D), jnp.float32),
    ),
    mesh=_mesh,
    scratch_types=[
        pltpu.VMEM((CH2,), jnp.int32),
        pltpu.VMEM((CH2,), jnp.int32),
        pltpu.VMEM((CH2, 2 * D), jnp.float32),
        pltpu.VMEM((CH2, D), jnp.float32),
        pltpu.VMEM((CH2, D), jnp.float32),
        pltpu.VMEM((CH2, D), jnp.float32),
        pltpu.VMEM((4, D), jnp.float32),
        pltpu.VMEM((ZCH, D), jnp.float32),
        pltpu.VMEM_SHARED((N, D), jnp.float32),
        pltpu.SemaphoreType.DMA,
        pltpu.SemaphoreType.DMA,
    ],
)(_pass2_body)


def kernel(edge_index, h, e, params):
    src = edge_index[0]
    dst = edge_index[1]
    for p in params:
        wt = jnp.concatenate(
            [p['A_w'], p['B_w'], p['Vf_w'], p['Vb_w'], p['U_w']], axis=0).T
        bb = jnp.concatenate(
            [p['A_b'], p['B_b'], p['Vf_b'], p['Vb_b'], p['U_b']])[None]
        t1, t2, u = _node_mm(h, wt, bb)
        c = _edge_mm(e, p['C_w'].T, p['C_b'][None])
        xf, xb, stats = _pass1(t1, src, dst, c)
        s = stats.sum(axis=0)
        mu_f = s[0] / E
        var_f = s[1] / E - mu_f * mu_f
        mu_b = s[2] / E
        var_b = s[3] / E - mu_b * mu_b
        scf = p['bn_e_g'] * lax.rsqrt(var_f + 1e-5)
        shf = p['bn_e_b'] - mu_f * scf
        scb = p['bn_e_g'] * lax.rsqrt(var_b + 1e-5)
        shb = p['bn_e_b'] - mu_b * scb
        ss = jnp.stack([scf, shf, scb, shb])
        nef, hp = _pass2(t2, src, dst, xf, xb, e, ss)
        h = _node_bn(hp, u, h, p['bn_h_g'][None], p['bn_h_b'][None])
        e = nef
    return h, e


# speedup vs baseline: 1.4324x; 1.1372x over previous
import functools

import jax
import jax.numpy as jnp
from jax import lax
from jax.experimental import pallas as pl
from jax.experimental.pallas import tpu as pltpu
from jax.experimental.pallas import tpu_sc as plsc

N = 10000
E = 320000
D = 128

NC = 2
NS = 16
NW = NC * NS
EPW = E // NW
CH1 = 40
NCK1 = EPW // CH1
CH2 = 80
NCK2 = EPW // CH2
ZCH = 80
NZC = N // ZCH
NLV = D // 16

_mesh = plsc.VectorSubcoreMesh(core_axis_name="c", subcore_axis_name="s")


def _node_mm_body(h_ref, w_ref, b_ref, t1_ref, t2_ref, u_ref):
    acc = jnp.dot(h_ref[...], w_ref[...], preferred_element_type=jnp.float32)
    acc = acc + b_ref[...]
    t1_ref[...] = acc[:, :2 * D]
    t2_ref[...] = acc[:, 2 * D:4 * D]
    u_ref[...] = acc[:, 4 * D:]


def _node_mm(h, wt, bb):
    bn = 2000
    return pl.pallas_call(
        _node_mm_body,
        grid=(N // bn,),
        in_specs=[
            pl.BlockSpec((bn, D), lambda i: (i, 0)),
            pl.BlockSpec((D, 5 * D), lambda i: (0, 0)),
            pl.BlockSpec((1, 5 * D), lambda i: (0, 0)),
        ],
        out_specs=[
            pl.BlockSpec((bn, 2 * D), lambda i: (i, 0)),
            pl.BlockSpec((bn, 2 * D), lambda i: (i, 0)),
            pl.BlockSpec((bn, D), lambda i: (i, 0)),
        ],
        out_shape=[
            jax.ShapeDtypeStruct((N, 2 * D), jnp.float32),
            jax.ShapeDtypeStruct((N, 2 * D), jnp.float32),
            jax.ShapeDtypeStruct((N, D), jnp.float32),
        ],
    )(h, wt, bb)


def _edge_mm_body(e_ref, w_ref, b_ref, c_ref):
    c_ref[...] = jnp.dot(e_ref[...], w_ref[...],
                         preferred_element_type=jnp.float32) + b_ref[...]


def _edge_mm(e, wt, bb):
    bn = 16000
    return pl.pallas_call(
        _edge_mm_body,
        grid=(E // bn,),
        in_specs=[
            pl.BlockSpec((bn, D), lambda i: (i, 0)),
            pl.BlockSpec((D, D), lambda i: (0, 0)),
            pl.BlockSpec((1, D), lambda i: (0, 0)),
        ],
        out_specs=pl.BlockSpec((bn, D), lambda i: (i, 0)),
        out_shape=jax.ShapeDtypeStruct((E, D), jnp.float32),
    )(e, wt, bb)


def _node_bn_body(hp_ref, u_ref, hin_ref, g_ref, b_ref, out_ref):
    s = hp_ref[0] + hp_ref[1] + u_ref[...]
    mu = jnp.mean(s, axis=0, keepdims=True)
    var = jnp.mean((s - mu) ** 2, axis=0, keepdims=True)
    y = (s - mu) * lax.rsqrt(var + 1e-5) * g_ref[...] + b_ref[...]
    out_ref[...] = jnp.maximum(y, 0.0) + hin_ref[...]


def _node_bn(hp, u, hin, g, b):
    return pl.pallas_call(
        _node_bn_body,
        out_shape=jax.ShapeDtypeStruct((N, D), jnp.float32),
    )(hp, u, hin, g, b)


def _pass1_body(t1_hbm, src_hbm, dst_hbm, c_hbm,
                xf_hbm, xb_hbm, stats_hbm,
                ei_v, rs_v, rd_v, c_v, xf_v, xb_v, stats_v,
                sei, sin, sout):
    cid = lax.axis_index("c")
    sid = lax.axis_index("s")
    wid = sid * NC + cid
    e0 = wid * EPW

    z = jnp.zeros((16,), jnp.float32)
    for r in range(4):
        for c in range(NLV):
            stats_v[r, pl.ds(c * 16, 16)] = z

    pltpu.async_copy(src_hbm.at[pl.ds(e0, CH1)], ei_v.at[0].at[0], sei.at[0])
    pltpu.async_copy(dst_hbm.at[pl.ds(e0, CH1)], ei_v.at[0].at[1], sei.at[0])

    def stage(i, b, ob):
        base = e0 + i * CH1
        pltpu.make_async_copy(src_hbm.at[pl.ds(0, CH1)], ei_v.at[b].at[0],
                              sei.at[b]).wait()
        pltpu.make_async_copy(dst_hbm.at[pl.ds(0, CH1)], ei_v.at[b].at[1],
                              sei.at[b]).wait()
        g1 = pltpu.async_copy(t1_hbm.at[ei_v.at[b].at[0]], rs_v.at[b], sin.at[b])
        g2 = pltpu.async_copy(t1_hbm.at[ei_v.at[b].at[1]], rd_v.at[b], sin.at[b])
        g3 = pltpu.async_copy(c_hbm.at[pl.ds(base, CH1)], c_v.at[b], sin.at[b])
        nxt = jnp.where(i + 1 < NCK1, i + 1, 0)
        pltpu.async_copy(src_hbm.at[pl.ds(e0 + nxt * CH1, CH1)],
                         ei_v.at[ob].at[0], sei.at[ob])
        pltpu.async_copy(dst_hbm.at[pl.ds(e0 + nxt * CH1, CH1)],
                         ei_v.at[ob].at[1], sei.at[ob])

        @pl.when(i >= 2)
        def _():
            pltpu.make_async_copy(xf_v.at[b], xf_hbm.at[pl.ds(0, CH1)],
                                  sout.at[b]).wait()
            pltpu.make_async_copy(xb_v.at[b], xb_hbm.at[pl.ds(0, CH1)],
                                  sout.at[b]).wait()

        g1.wait()
        g2.wait()
        g3.wait()

        def row(j, carry):
            acc = list(carry)
            for c in range(NLV):
                sl = pl.ds(c * 16, 16)
                sl2 = pl.ds(D + c * 16, 16)
                a_s = rs_v[b, j, sl]
                b_s = rs_v[b, j, sl2]
                a_d = rd_v[b, j, sl]
                b_d = rd_v[b, j, sl2]
                cc = c_v[b, j, sl]
                xf = a_s + b_d + cc
                xb = a_d + b_s + cc
                xf_v[b, j, sl] = xf
                xb_v[b, j, sl] = xb
                acc[c] = acc[c] + xf
                acc[NLV + c] = acc[NLV + c] + xf * xf
                acc[2 * NLV + c] = acc[2 * NLV + c] + xb
                acc[3 * NLV + c] = acc[3 * NLV + c] + xb * xb
            return tuple(acc)

        carry = lax.fori_loop(0, CH1, row, (z,) * (4 * NLV))
        for c in range(NLV):
            sl = pl.ds(c * 16, 16)
            stats_v[0, sl] += carry[c]
            stats_v[1, sl] += carry[NLV + c]
            stats_v[2, sl] += carry[2 * NLV + c]
            stats_v[3, sl] += carry[3 * NLV + c]
        pltpu.async_copy(xf_v.at[b], xf_hbm.at[pl.ds(base, CH1)], sout.at[b])
        pltpu.async_copy(xb_v.at[b], xb_hbm.at[pl.ds(base, CH1)], sout.at[b])

    @pl.loop(0, NCK1 // 2)
    def _grp(g):
        stage(2 * g, 0, 1)
        stage(2 * g + 1, 1, 0)

    pltpu.make_async_copy(src_hbm.at[pl.ds(0, CH1)], ei_v.at[0].at[0],
                          sei.at[0]).wait()
    pltpu.make_async_copy(dst_hbm.at[pl.ds(0, CH1)], ei_v.at[0].at[1],
                          sei.at[0]).wait()
    for b in range(2):
        pltpu.make_async_copy(xf_v.at[b], xf_hbm.at[pl.ds(0, CH1)],
                              sout.at[b]).wait()
        pltpu.make_async_copy(xb_v.at[b], xb_hbm.at[pl.ds(0, CH1)],
                              sout.at[b]).wait()
    pltpu.sync_copy(stats_v, stats_hbm.at[wid])


_pass1 = functools.partial(
    pl.kernel,
    out_type=(
        jax.ShapeDtypeStruct((E, D), jnp.float32),
        jax.ShapeDtypeStruct((E, D), jnp.float32),
        jax.ShapeDtypeStruct((NW, 4, D), jnp.float32),
    ),
    mesh=_mesh,
    scratch_types=[
        pltpu.VMEM((2, 2, CH1), jnp.int32),
        pltpu.VMEM((2, CH1, 2 * D), jnp.float32),
        pltpu.VMEM((2, CH1, 2 * D), jnp.float32),
        pltpu.VMEM((2, CH1, D), jnp.float32),
        pltpu.VMEM((2, CH1, D), jnp.float32),
        pltpu.VMEM((2, CH1, D), jnp.float32),
        pltpu.VMEM((4, D), jnp.float32),
        pltpu.SemaphoreType.DMA((2,)),
        pltpu.SemaphoreType.DMA((2,)),
        pltpu.SemaphoreType.DMA((2,)),
    ],
)(_pass1_body)


def _pass2a_body(t2_hbm, src_hbm, xf_hbm, xb_hbm, e_hbm, ss_hbm,
                 nef_hbm, msg_hbm,
                 ei_v, r2_v, xf_v, xb_v, e_v, ss_v,
                 sei, sin, sout):
    cid = lax.axis_index("c")
    sid = lax.axis_index("s")
    wid = sid * NC + cid
    e0 = wid * EPW

    pltpu.sync_copy(ss_hbm, ss_v)
    scf = [ss_v[0, pl.ds(c * 16, 16)] for c in range(NLV)]
    shf = [ss_v[1, pl.ds(c * 16, 16)] for c in range(NLV)]
    scb = [ss_v[2, pl.ds(c * 16, 16)] for c in range(NLV)]
    shb = [ss_v[3, pl.ds(c * 16, 16)] for c in range(NLV)]

    pltpu.async_copy(src_hbm.at[pl.ds(e0, CH2)], ei_v.at[0], sei.at[0])

    def stage(i, b, ob):
        base = e0 + i * CH2
        pltpu.make_async_copy(src_hbm.at[pl.ds(0, CH2)], ei_v.at[b],
                              sei.at[b]).wait()
        g1 = pltpu.async_copy(t2_hbm.at[ei_v.at[b]], r2_v.at[b], sin.at[b])
        g2 = pltpu.async_copy(xf_hbm.at[pl.ds(base, CH2)], xf_v.at[b], sin.at[b])
        g3 = pltpu.async_copy(xb_hbm.at[pl.ds(base, CH2)], xb_v.at[b], sin.at[b])
        g4 = pltpu.async_copy(e_hbm.at[pl.ds(base, CH2)], e_v.at[b], sin.at[b])
        nxt = jnp.where(i + 1 < NCK2, i + 1, 0)
        pltpu.async_copy(src_hbm.at[pl.ds(e0 + nxt * CH2, CH2)],
                         ei_v.at[ob], sei.at[ob])

        @pl.when(i >= 2)
        def _():
            pltpu.make_async_copy(xf_v.at[b], nef_hbm.at[pl.ds(0, CH2)],
                                  sout.at[b]).wait()
            pltpu.make_async_copy(e_v.at[b], msg_hbm.at[pl.ds(0, CH2)],
                                  sout.at[b]).wait()

        g1.wait()
        g2.wait()
        g3.wait()
        g4.wait()

        def row(j, _):
            for c in range(NLV):
                sl = pl.ds(c * 16, 16)
                sl2 = pl.ds(D + c * 16, 16)
                ee = e_v[b, j, sl]
                nf = jnp.maximum(xf_v[b, j, sl] * scf[c] + shf[c], 0.0) + ee
                nb = jnp.maximum(xb_v[b, j, sl] * scb[c] + shb[c], 0.0) + ee
                xf_v[b, j, sl] = nf
                gf = 1.0 / (1e-6 * jnp.exp(-nf) + (1.0 + 1e-6))
                gb = 1.0 / (1e-6 * jnp.exp(-nb) + (1.0 + 1e-6))
                e_v[b, j, sl] = r2_v[b, j, sl] * gf + r2_v[b, j, sl2] * gb
            return 0

        lax.fori_loop(0, CH2, row, 0)
        pltpu.async_copy(xf_v.at[b], nef_hbm.at[pl.ds(base, CH2)], sout.at[b])
        pltpu.async_copy(e_v.at[b], msg_hbm.at[pl.ds(base, CH2)], sout.at[b])

    @pl.loop(0, NCK2 // 2)
    def _grp(g):
        stage(2 * g, 0, 1)
        stage(2 * g + 1, 1, 0)

    if NCK2 % 2:
        stage(NCK2 - 1, 0, 1)

    pltpu.make_async_copy(src_hbm.at[pl.ds(0, CH2)], ei_v.at[NCK2 % 2],
                          sei.at[NCK2 % 2]).wait()
    for b in range(2):
        pltpu.make_async_copy(xf_v.at[b], nef_hbm.at[pl.ds(0, CH2)],
                              sout.at[b]).wait()
        pltpu.make_async_copy(e_v.at[b], msg_hbm.at[pl.ds(0, CH2)],
                              sout.at[b]).wait()


_pass2a = functools.partial(
    pl.kernel,
    out_type=(
        jax.ShapeDtypeStruct((E, D), jnp.float32),
        jax.ShapeDtypeStruct((E, D), jnp.float32),
    ),
    mesh=_mesh,
    scratch_types=[
        pltpu.VMEM((2, CH2), jnp.int32),
        pltpu.VMEM((2, CH2, 2 * D), jnp.float32),
        pltpu.VMEM((2, CH2, D), jnp.float32),
        pltpu.VMEM((2, CH2, D), jnp.float32),
        pltpu.VMEM((2, CH2, D), jnp.float32),
        pltpu.VMEM((4, D), jnp.float32),
        pltpu.SemaphoreType.DMA((2,)),
        pltpu.SemaphoreType.DMA((2,)),
        pltpu.SemaphoreType.DMA((2,)),
    ],
)(_pass2a_body)


def _pass2b_body(dst_hbm, msg_hbm, hp_hbm,
                 ei_v, msg_v, acc_sh, sei, sin):
    cid = lax.axis_index("c")
    sid = lax.axis_index("s")
    wid = sid * NC + cid
    e0 = wid * EPW

    z = jnp.zeros((16,), jnp.float32)

    def zrow(j, _):
        for c in range(NLV):
            msg_v[0, j, pl.ds(c * 16, 16)] = z
        return 0

    lax.fori_loop(0, ZCH, zrow, 0)
    for k in range((NZC + NS - 1) // NS):
        ci = sid + k * NS

        @pl.when(ci < NZC)
        def _():
            pltpu.sync_copy(msg_v.at[0], acc_sh.at[pl.ds(ci * ZCH, ZCH)])

    plsc.subcore_barrier()

    pltpu.async_copy(dst_hbm.at[pl.ds(e0, CH2)], ei_v.at[0], sei.at[0])
    pltpu.async_copy(msg_hbm.at[pl.ds(e0, CH2)], msg_v.at[0], sin.at[0])

    def stage(i, b, ob):
        nxt = jnp.where(i + 1 < NCK2, i + 1, 0)
        pltpu.async_copy(dst_hbm.at[pl.ds(e0 + nxt * CH2, CH2)],
                         ei_v.at[ob], sei.at[ob])
        pltpu.async_copy(msg_hbm.at[pl.ds(e0 + nxt * CH2, CH2)],
                         msg_v.at[ob], sin.at[ob])
        pltpu.make_async_copy(dst_hbm.at[pl.ds(0, CH2)], ei_v.at[b],
                              sei.at[b]).wait()
        pltpu.make_async_copy(msg_hbm.at[pl.ds(0, CH2)], msg_v.at[b],
                              sin.at[b]).wait()
        pltpu.sync_copy(msg_v.at[b], acc_sh.at[ei_v.at[b]], add=True)

    @pl.loop(0, NCK2 // 2)
    def _grp(g):
        stage(2 * g, 0, 1)
        stage(2 * g + 1, 1, 0)

    if NCK2 % 2:
        stage(NCK2 - 1, 0, 1)

    pltpu.make_async_copy(dst_hbm.at[pl.ds(0, CH2)], ei_v.at[NCK2 % 2],
                          sei.at[NCK2 % 2]).wait()
    pltpu.make_async_copy(msg_hbm.at[pl.ds(0, CH2)], msg_v.at[NCK2 % 2],
                          sin.at[NCK2 % 2]).wait()

    plsc.subcore_barrier()
    for k in range((NZC + NS - 1) // NS):
        ci = sid + k * NS

        @pl.when(ci < NZC)
        def _():
            r0 = ci * ZCH
            pltpu.sync_copy(acc_sh.at[pl.ds(r0, ZCH)],
                            hp_hbm.at[cid, pl.ds(r0, ZCH)])


_pass2b = functools.partial(
    pl.kernel,
    out_type=jax.ShapeDtypeStruct((NC, N, D), jnp.float32),
    mesh=_mesh,
    scratch_types=[
        pltpu.VMEM((2, CH2), jnp.int32),
        pltpu.VMEM((2, CH2, D), jnp.float32),
        pltpu.VMEM_SHARED((N, D), jnp.float32),
        pltpu.SemaphoreType.DMA((2,)),
        pltpu.SemaphoreType.DMA((2,)),
    ],
)(_pass2b_body)


def kernel(edge_index, h, e, params):
    for p in params:
        wt = jnp.concatenate(
            [p['A_w'], p['B_w'], p['Vf_w'], p['Vb_w'], p['U_w']], axis=0).T
        bb = jnp.concatenate(
            [p['A_b'], p['B_b'], p['Vf_b'], p['Vb_b'], p['U_b']])[None]
        t1, t2, u = _node_mm(h, wt, bb)
        c = _edge_mm(e, p['C_w'].T, p['C_b'][None])
        xf, xb, stats = _pass1(t1, edge_index[0], edge_index[1], c)
        s = stats.sum(axis=0)
        mu_f = s[0] / E
        var_f = s[1] / E - mu_f * mu_f
        mu_b = s[2] / E
        var_b = s[3] / E - mu_b * mu_b
        scf = p['bn_e_g'] * lax.rsqrt(var_f + 1e-5)
        shf = p['bn_e_b'] - mu_f * scf
        scb = p['bn_e_g'] * lax.rsqrt(var_b + 1e-5)
        shb = p['bn_e_b'] - mu_b * scb
        ss = jnp.stack([scf, shf, scb, shb])
        nef, msg = _pass2a(t2, edge_index[0], xf, xb, e, ss)
        hp = _pass2b(edge_index[1], msg)
        h = _node_bn(hp, u, h, p['bn_h_g'][None], p['bn_h_b'][None])
        e = nef
    return h, e


# speedup vs baseline: 3.6344x; 2.5373x over previous
import functools

import jax
import jax.numpy as jnp
from jax import lax
from jax.experimental import pallas as pl
from jax.experimental.pallas import tpu as pltpu
from jax.experimental.pallas import tpu_sc as plsc

N = 10000
E = 320000
D = 128

NC = 2
NS = 16
NW = NC * NS
EPW = E // NW
CH1 = 40
NCK1 = EPW // CH1
CH2 = 80
NCK2 = EPW // CH2
ZCH = 80
NZC = N // ZCH
NLV = D // 16

_mesh = plsc.VectorSubcoreMesh(core_axis_name="c", subcore_axis_name="s")


def _node_mm_body(h_ref, w_ref, b_ref, t1_ref, t2_ref, u_ref):
    acc = jnp.dot(h_ref[...], w_ref[...], preferred_element_type=jnp.float32)
    acc = acc + b_ref[...]
    t1_ref[...] = acc[:, :2 * D]
    t2_ref[...] = acc[:, 2 * D:4 * D]
    u_ref[...] = acc[:, 4 * D:]


def _node_mm(h, wt, bb):
    bn = 2000
    return pl.pallas_call(
        _node_mm_body,
        grid=(N // bn,),
        in_specs=[
            pl.BlockSpec((bn, D), lambda i: (i, 0)),
            pl.BlockSpec((D, 5 * D), lambda i: (0, 0)),
            pl.BlockSpec((1, 5 * D), lambda i: (0, 0)),
        ],
        out_specs=[
            pl.BlockSpec((bn, 2 * D), lambda i: (i, 0)),
            pl.BlockSpec((bn, 2 * D), lambda i: (i, 0)),
            pl.BlockSpec((bn, D), lambda i: (i, 0)),
        ],
        out_shape=[
            jax.ShapeDtypeStruct((N, 2 * D), jnp.float32),
            jax.ShapeDtypeStruct((N, 2 * D), jnp.float32),
            jax.ShapeDtypeStruct((N, D), jnp.float32),
        ],
    )(h, wt, bb)


def _edge_mm_body(e_ref, w_ref, b_ref, c_ref):
    c_ref[...] = jnp.dot(e_ref[...], w_ref[...],
                         preferred_element_type=jnp.float32) + b_ref[...]


def _edge_mm(e, wt, bb):
    bn = 16000
    return pl.pallas_call(
        _edge_mm_body,
        grid=(E // bn,),
        in_specs=[
            pl.BlockSpec((bn, D), lambda i: (i, 0)),
            pl.BlockSpec((D, D), lambda i: (0, 0)),
            pl.BlockSpec((1, D), lambda i: (0, 0)),
        ],
        out_specs=pl.BlockSpec((bn, D), lambda i: (i, 0)),
        out_shape=jax.ShapeDtypeStruct((E, D), jnp.float32),
    )(e, wt, bb)


def _node_bn_body(hp_ref, u_ref, hin_ref, g_ref, b_ref, out_ref):
    s = hp_ref[0] + hp_ref[1] + u_ref[...]
    mu = jnp.mean(s, axis=0, keepdims=True)
    var = jnp.mean((s - mu) ** 2, axis=0, keepdims=True)
    y = (s - mu) * lax.rsqrt(var + 1e-5) * g_ref[...] + b_ref[...]
    out_ref[...] = jnp.maximum(y, 0.0) + hin_ref[...]


def _node_bn(hp, u, hin, g, b):
    return pl.pallas_call(
        _node_bn_body,
        out_shape=jax.ShapeDtypeStruct((N, D), jnp.float32),
    )(hp, u, hin, g, b)


def _pass1_body(t1_hbm, ei4_hbm, c_hbm,
                xf_hbm, xb_hbm, stats_hbm,
                eib, rs_v, c_v, xf_v, xb_v, stats_v,
                sin, sout):
    cid = lax.axis_index("c")
    sid = lax.axis_index("s")
    wid = sid * NC + cid
    e0 = wid * EPW

    z = jnp.zeros((16,), jnp.float32)
    for r in range(4):
        for c in range(NLV):
            stats_v[r, pl.ds(c * 16, 16)] = z

    pltpu.sync_copy(ei4_hbm.at[wid], eib)

    def issue(i, b):
        pltpu.async_copy(t1_hbm.at[eib.at[i]], rs_v.at[b], sin.at[b])
        pltpu.async_copy(c_hbm.at[pl.ds(e0 + i * CH1, CH1)], c_v.at[b],
                         sin.at[b])

    issue(0, 0)

    def stage(i, b, ob):
        @pl.when(i + 1 < NCK1)
        def _():
            issue(i + 1, ob)

        @pl.when(i >= 2)
        def _():
            pltpu.make_async_copy(xf_v.at[b], xf_hbm.at[pl.ds(0, CH1)],
                                  sout.at[b]).wait()
            pltpu.make_async_copy(xb_v.at[b], xb_hbm.at[pl.ds(0, CH1)],
                                  sout.at[b]).wait()

        pltpu.make_async_copy(t1_hbm.at[eib.at[0]], rs_v.at[b],
                              sin.at[b]).wait()
        pltpu.make_async_copy(c_hbm.at[pl.ds(0, CH1)], c_v.at[b],
                              sin.at[b]).wait()

        def row(j, carry):
            acc = list(carry)
            for c in range(NLV):
                sl = pl.ds(c * 16, 16)
                sl2 = pl.ds(D + c * 16, 16)
                a_s = rs_v[b, j, sl]
                b_s = rs_v[b, j, sl2]
                a_d = rs_v[b, CH1 + j, sl]
                b_d = rs_v[b, CH1 + j, sl2]
                cc = c_v[b, j, sl]
                xf = a_s + b_d + cc
                xb = a_d + b_s + cc
                xf_v[b, j, sl] = xf
                xb_v[b, j, sl] = xb
                acc[c] = acc[c] + xf
                acc[NLV + c] = acc[NLV + c] + xf * xf
                acc[2 * NLV + c] = acc[2 * NLV + c] + xb
                acc[3 * NLV + c] = acc[3 * NLV + c] + xb * xb
            return tuple(acc)

        carry = lax.fori_loop(0, CH1, row, (z,) * (4 * NLV))
        for c in range(NLV):
            sl = pl.ds(c * 16, 16)
            stats_v[0, sl] += carry[c]
            stats_v[1, sl] += carry[NLV + c]
            stats_v[2, sl] += carry[2 * NLV + c]
            stats_v[3, sl] += carry[3 * NLV + c]
        base = e0 + i * CH1
        pltpu.async_copy(xf_v.at[b], xf_hbm.at[pl.ds(base, CH1)], sout.at[b])
        pltpu.async_copy(xb_v.at[b], xb_hbm.at[pl.ds(base, CH1)], sout.at[b])

    @pl.loop(0, NCK1 // 2)
    def _grp(g):
        stage(2 * g, 0, 1)
        stage(2 * g + 1, 1, 0)

    for b in range(2):
        pltpu.make_async_copy(xf_v.at[b], xf_hbm.at[pl.ds(0, CH1)],
                              sout.at[b]).wait()
        pltpu.make_async_copy(xb_v.at[b], xb_hbm.at[pl.ds(0, CH1)],
                              sout.at[b]).wait()
    pltpu.sync_copy(stats_v, stats_hbm.at[wid])


_pass1 = functools.partial(
    pl.kernel,
    out_type=(
        jax.ShapeDtypeStruct((E, D), jnp.float32),
        jax.ShapeDtypeStruct((E, D), jnp.float32),
        jax.ShapeDtypeStruct((NW, 4, D), jnp.float32),
    ),
    mesh=_mesh,
    scratch_types=[
        pltpu.VMEM((NCK1, 2 * CH1), jnp.int32),
        pltpu.VMEM((2, 2 * CH1, 2 * D), jnp.float32),
        pltpu.VMEM((2, CH1, D), jnp.float32),
        pltpu.VMEM((2, CH1, D), jnp.float32),
        pltpu.VMEM((2, CH1, D), jnp.float32),
        pltpu.VMEM((4, D), jnp.float32),
        pltpu.SemaphoreType.DMA((2,)),
        pltpu.SemaphoreType.DMA((2,)),
    ],
)(_pass1_body)


def _gate_body(xf_ref, xb_ref, e_ref, ss_ref, nef_ref, gf_ref, gb_ref):
    ss = ss_ref[...]
    ee = e_ref[...]
    nf = jnp.maximum(xf_ref[...] * ss[0:1] + ss[1:2], 0.0) + ee
    nb = jnp.maximum(xb_ref[...] * ss[2:3] + ss[3:4], 0.0) + ee
    nef_ref[...] = nf
    gf_ref[...] = 1.0 / (1e-6 * jnp.exp(-nf) + (1.0 + 1e-6))
    gb_ref[...] = 1.0 / (1e-6 * jnp.exp(-nb) + (1.0 + 1e-6))


def _gate(xf, xb, e, ss):
    bn = 4000
    spec = pl.BlockSpec((bn, D), lambda i: (i, 0))
    return pl.pallas_call(
        _gate_body,
        grid=(E // bn,),
        in_specs=[spec, spec, spec, pl.BlockSpec((4, D), lambda i: (0, 0))],
        out_specs=[spec, spec, spec],
        out_shape=[
            jax.ShapeDtypeStruct((E, D), jnp.float32),
            jax.ShapeDtypeStruct((E, D), jnp.float32),
            jax.ShapeDtypeStruct((E, D), jnp.float32),
        ],
    )(xf, xb, e, ss)


def _pass2a_body(t2_hbm, src3_hbm, gf_hbm, gb_hbm,
                 msg_hbm,
                 srcb, r2_v, gf_v, gb_v,
                 sin, sout):
    cid = lax.axis_index("c")
    sid = lax.axis_index("s")
    wid = sid * NC + cid
    e0 = wid * EPW

    pltpu.sync_copy(src3_hbm.at[wid], srcb)

    def issue(i, b):
        base = e0 + i * CH2
        pltpu.async_copy(t2_hbm.at[srcb.at[i]], r2_v.at[b], sin.at[b])
        pltpu.async_copy(gf_hbm.at[pl.ds(base, CH2)], gf_v.at[b], sin.at[b])
        pltpu.async_copy(gb_hbm.at[pl.ds(base, CH2)], gb_v.at[b], sin.at[b])

    issue(0, 0)

    def stage(i, b, ob):
        @pl.when(i + 1 < NCK2)
        def _():
            issue(i + 1, ob)

        @pl.when(i >= 2)
        def _():
            pltpu.make_async_copy(gf_v.at[b], msg_hbm.at[pl.ds(0, CH2)],
                                  sout.at[b]).wait()

        pltpu.make_async_copy(t2_hbm.at[srcb.at[0]], r2_v.at[b],
                              sin.at[b]).wait()
        pltpu.make_async_copy(gf_hbm.at[pl.ds(0, CH2)], gf_v.at[b],
                              sin.at[b]).wait()
        pltpu.make_async_copy(gb_hbm.at[pl.ds(0, CH2)], gb_v.at[b],
                              sin.at[b]).wait()

        def row(j, _):
            for c in range(NLV):
                sl = pl.ds(c * 16, 16)
                sl2 = pl.ds(D + c * 16, 16)
                gf_v[b, j, sl] = (r2_v[b, j, sl] * gf_v[b, j, sl]
                                  + r2_v[b, j, sl2] * gb_v[b, j, sl])
            return 0

        lax.fori_loop(0, CH2, row, 0)
        pltpu.async_copy(gf_v.at[b], msg_hbm.at[pl.ds(e0 + i * CH2, CH2)],
                         sout.at[b])

    @pl.loop(0, NCK2 // 2)
    def _grp(g):
        stage(2 * g, 0, 1)
        stage(2 * g + 1, 1, 0)

    if NCK2 % 2:
        stage(NCK2 - 1, 0, 1)

    for b in range(2):
        pltpu.make_async_copy(gf_v.at[b], msg_hbm.at[pl.ds(0, CH2)],
                              sout.at[b]).wait()


_pass2a = functools.partial(
    pl.kernel,
    out_type=jax.ShapeDtypeStruct((E, D), jnp.float32),
    mesh=_mesh,
    scratch_types=[
        pltpu.VMEM((NCK2, CH2), jnp.int32),
        pltpu.VMEM((2, CH2, 2 * D), jnp.float32),
        pltpu.VMEM((2, CH2, D), jnp.float32),
        pltpu.VMEM((2, CH2, D), jnp.float32),
        pltpu.SemaphoreType.DMA((2,)),
        pltpu.SemaphoreType.DMA((2,)),
    ],
)(_pass2a_body)


def _pass2b_body(dst_hbm, msg_hbm, hp_hbm,
                 ei_v, msg_v, acc_sh, sei, sin):
    cid = lax.axis_index("c")
    sid = lax.axis_index("s")
    wid = sid * NC + cid
    e0 = wid * EPW

    z = jnp.zeros((16,), jnp.float32)

    def zrow(j, _):
        for c in range(NLV):
            msg_v[0, j, pl.ds(c * 16, 16)] = z
        return 0

    lax.fori_loop(0, ZCH, zrow, 0)
    for k in range((NZC + NS - 1) // NS):
        ci = sid + k * NS

        @pl.when(ci < NZC)
        def _():
            pltpu.sync_copy(msg_v.at[0], acc_sh.at[pl.ds(ci * ZCH, ZCH)])

    plsc.subcore_barrier()

    pltpu.async_copy(dst_hbm.at[pl.ds(e0, CH2)], ei_v.at[0], sei.at[0])
    pltpu.async_copy(msg_hbm.at[pl.ds(e0, CH2)], msg_v.at[0], sin.at[0])

    def stage(i, b, ob):
        nxt = jnp.where(i + 1 < NCK2, i + 1, 0)
        pltpu.async_copy(dst_hbm.at[pl.ds(e0 + nxt * CH2, CH2)],
                         ei_v.at[ob], sei.at[ob])
        pltpu.async_copy(msg_hbm.at[pl.ds(e0 + nxt * CH2, CH2)],
                         msg_v.at[ob], sin.at[ob])
        pltpu.make_async_copy(dst_hbm.at[pl.ds(0, CH2)], ei_v.at[b],
                              sei.at[b]).wait()
        pltpu.make_async_copy(msg_hbm.at[pl.ds(0, CH2)], msg_v.at[b],
                              sin.at[b]).wait()
        pltpu.sync_copy(msg_v.at[b], acc_sh.at[ei_v.at[b]], add=True)

    @pl.loop(0, NCK2 // 2)
    def _grp(g):
        stage(2 * g, 0, 1)
        stage(2 * g + 1, 1, 0)

    if NCK2 % 2:
        stage(NCK2 - 1, 0, 1)

    pltpu.make_async_copy(dst_hbm.at[pl.ds(0, CH2)], ei_v.at[NCK2 % 2],
                          sei.at[NCK2 % 2]).wait()
    pltpu.make_async_copy(msg_hbm.at[pl.ds(0, CH2)], msg_v.at[NCK2 % 2],
                          sin.at[NCK2 % 2]).wait()

    plsc.subcore_barrier()
    for k in range((NZC + NS - 1) // NS):
        ci = sid + k * NS

        @pl.when(ci < NZC)
        def _():
            r0 = ci * ZCH
            pltpu.sync_copy(acc_sh.at[pl.ds(r0, ZCH)],
                            hp_hbm.at[cid, pl.ds(r0, ZCH)])


_pass2b = functools.partial(
    pl.kernel,
    out_type=jax.ShapeDtypeStruct((NC, N, D), jnp.float32),
    mesh=_mesh,
    scratch_types=[
        pltpu.VMEM((2, CH2), jnp.int32),
        pltpu.VMEM((2, CH2, D), jnp.float32),
        pltpu.VMEM_SHARED((N, D), jnp.float32),
        pltpu.SemaphoreType.DMA((2,)),
        pltpu.SemaphoreType.DMA((2,)),
    ],
)(_pass2b_body)


def kernel(edge_index, h, e, params):
    src_i = edge_index[0]
    dst_i = edge_index[1]
    src3 = src_i.reshape(NW, NCK2, CH2)
    ei4 = jnp.concatenate([src_i.reshape(NW, NCK1, CH1),
                           dst_i.reshape(NW, NCK1, CH1)], axis=2)
    for p in params:
        wt = jnp.concatenate(
            [p['A_w'], p['B_w'], p['Vf_w'], p['Vb_w'], p['U_w']], axis=0).T
        bb = jnp.concatenate(
            [p['A_b'], p['B_b'], p['Vf_b'], p['Vb_b'], p['U_b']])[None]
        t1, t2, u = _node_mm(h, wt, bb)
        c = _edge_mm(e, p['C_w'].T, p['C_b'][None])
        xf, xb, stats = _pass1(t1, ei4, c)
        s = stats.sum(axis=0)
        mu_f = s[0] / E
        var_f = s[1] / E - mu_f * mu_f
        mu_b = s[2] / E
        var_b = s[3] / E - mu_b * mu_b
        scf = p['bn_e_g'] * lax.rsqrt(var_f + 1e-5)
        shf = p['bn_e_b'] - mu_f * scf
        scb = p['bn_e_g'] * lax.rsqrt(var_b + 1e-5)
        shb = p['bn_e_b'] - mu_b * scb
        ss = jnp.stack([scf, shf, scb, shb])
        nef, gf, gb = _gate(xf, xb, e, ss)
        msg = _pass2a(t2, src3, gf, gb)
        hp = _pass2b(dst_i, msg)
        h = _node_bn(hp, u, h, p['bn_h_g'][None], p['bn_h_b'][None])
        e = nef
    return h, e
